# Initial kernel scaffold; baseline (speedup 1.0000x reference)
import jax
import jax.numpy as jnp
from jax.experimental import pallas as pl


def kernel(x, spec_router_logits, shared_router_logits, spec_expert_weights, shared_expert_weights):
    raise NotImplementedError("write your pallas kernel here")


# speedup vs baseline: 1.4026x; 1.4026x over previous
import jax
import jax.numpy as jnp
from jax.experimental import pallas as pl
from jax.experimental.pallas import tpu as pltpu

N_TOK = 4096
D_MODEL = 1024
NUM_SPEC = 8
NUM_SHARED = 2
NUM_TOTAL = NUM_SPEC + NUM_SHARED
BT = 1024


def _moe_body(sl_ref, shl_ref, x_ref, w_ref, o_ref, c_ref):
    e = pl.program_id(1)

    @pl.when(e == 0)
    def _compute_routing():
        sl = sl_ref[...]
        m = jnp.max(sl, axis=1, keepdims=True)
        ex = jnp.exp(sl - m)
        z = jnp.sum(ex, axis=1, keepdims=True)
        iota8 = jax.lax.broadcasted_iota(jnp.int32, sl.shape, 1)
        g1 = jnp.max(ex, axis=1, keepdims=True)
        a1 = jnp.min(jnp.where(ex == g1, iota8, NUM_SPEC), axis=1, keepdims=True)
        ex2 = jnp.where(iota8 == a1, 0.0, ex)
        g2 = jnp.max(ex2, axis=1, keepdims=True)
        a2 = jnp.min(jnp.where(ex2 == g2, iota8, NUM_SPEC), axis=1, keepdims=True)
        denom = g1 + g2 + 1e-6 * z
        w1 = g1 / denom
        w2 = g2 / denom
        lane = jax.lax.broadcasted_iota(jnp.int32, (sl.shape[0], 16), 1)
        c = w1 * (lane == a1).astype(jnp.float32) + w2 * (lane == a2).astype(jnp.float32)
        shl = shl_ref[...]
        sm = jnp.max(shl, axis=1, keepdims=True)
        sex = jnp.exp(shl - sm)
        ssum = jnp.sum(sex, axis=1, keepdims=True)
        c = c + (sex[:, 0:1] / ssum) * (lane == NUM_SPEC).astype(jnp.float32)
        c = c + (sex[:, 1:2] / ssum) * (lane == NUM_SPEC + 1).astype(jnp.float32)
        c_ref[...] = c

    lane = jax.lax.broadcasted_iota(jnp.int32, (BT, 16), 1)
    scale = jnp.sum(c_ref[...] * (lane == e).astype(jnp.float32), axis=1, keepdims=True)
    contrib = scale * jnp.dot(x_ref[...], w_ref[0], preferred_element_type=jnp.float32)

    @pl.when(e == 0)
    def _init():
        o_ref[...] = contrib

    @pl.when(e > 0)
    def _acc():
        o_ref[...] += contrib


def kernel(x, spec_router_logits, shared_router_logits, spec_expert_weights, shared_expert_weights):
    w_all = jnp.concatenate([spec_expert_weights, shared_expert_weights], axis=0).astype(jnp.bfloat16)
    x16 = x.astype(jnp.bfloat16)
    grid = (N_TOK // BT, NUM_TOTAL)
    return pl.pallas_call(
        _moe_body,
        grid=grid,
        in_specs=[
            pl.BlockSpec((BT, NUM_SPEC), lambda t, e: (t, 0)),
            pl.BlockSpec((BT, NUM_SHARED), lambda t, e: (t, 0)),
            pl.BlockSpec((BT, D_MODEL), lambda t, e: (t, 0)),
            pl.BlockSpec((1, D_MODEL, D_MODEL), lambda t, e: (e, 0, 0)),
        ],
        out_specs=pl.BlockSpec((BT, D_MODEL), lambda t, e: (t, 0)),
        out_shape=jax.ShapeDtypeStruct((N_TOK, D_MODEL), jnp.float32),
        scratch_shapes=[pltpu.VMEM((BT, 16), jnp.float32)],
        compiler_params=pltpu.CompilerParams(
            dimension_semantics=("parallel", "arbitrary"),
        ),
    )(spec_router_logits, shared_router_logits, x16, w_all)


# speedup vs baseline: 1.4152x; 1.0090x over previous
import jax
import jax.numpy as jnp
from jax.experimental import pallas as pl
from jax.experimental.pallas import tpu as pltpu

N_TOK = 4096
D_MODEL = 1024
NUM_SPEC = 8
NUM_SHARED = 2
NUM_TOTAL = NUM_SPEC + NUM_SHARED
BT = 2048


def _moe_body(sl_ref, shl_ref, x_ref, w_ref, o_ref, c_ref):
    e = pl.program_id(1)

    @pl.when(e == 0)
    def _compute_routing():
        sl = sl_ref[...]
        m = jnp.max(sl, axis=1, keepdims=True)
        ex = jnp.exp(sl - m)
        z = jnp.sum(ex, axis=1, keepdims=True)
        iota8 = jax.lax.broadcasted_iota(jnp.int32, sl.shape, 1)
        g1 = jnp.max(ex, axis=1, keepdims=True)
        a1 = jnp.min(jnp.where(ex == g1, iota8, NUM_SPEC), axis=1, keepdims=True)
        ex2 = jnp.where(iota8 == a1, 0.0, ex)
        g2 = jnp.max(ex2, axis=1, keepdims=True)
        a2 = jnp.min(jnp.where(ex2 == g2, iota8, NUM_SPEC), axis=1, keepdims=True)
        denom = g1 + g2 + 1e-6 * z
        w1 = g1 / denom
        w2 = g2 / denom
        lane = jax.lax.broadcasted_iota(jnp.int32, (sl.shape[0], 16), 1)
        c = w1 * (lane == a1).astype(jnp.float32) + w2 * (lane == a2).astype(jnp.float32)
        shl = shl_ref[...]
        sm = jnp.max(shl, axis=1, keepdims=True)
        sex = jnp.exp(shl - sm)
        ssum = jnp.sum(sex, axis=1, keepdims=True)
        c = c + (sex[:, 0:1] / ssum) * (lane == NUM_SPEC).astype(jnp.float32)
        c = c + (sex[:, 1:2] / ssum) * (lane == NUM_SPEC + 1).astype(jnp.float32)
        c_ref[...] = c

    lane = jax.lax.broadcasted_iota(jnp.int32, (BT, 16), 1)
    scale = jnp.sum(c_ref[...] * (lane == e).astype(jnp.float32), axis=1, keepdims=True)
    contrib = scale * jnp.dot(x_ref[...], w_ref[0], preferred_element_type=jnp.float32)

    @pl.when(e == 0)
    def _init():
        o_ref[...] = contrib

    @pl.when(e > 0)
    def _acc():
        o_ref[...] += contrib


def kernel(x, spec_router_logits, shared_router_logits, spec_expert_weights, shared_expert_weights):
    w_all = jnp.concatenate([spec_expert_weights, shared_expert_weights], axis=0).astype(jnp.bfloat16)
    x16 = x.astype(jnp.bfloat16)
    grid = (N_TOK // BT, NUM_TOTAL)
    return pl.pallas_call(
        _moe_body,
        grid=grid,
        in_specs=[
            pl.BlockSpec((BT, NUM_SPEC), lambda t, e: (t, 0)),
            pl.BlockSpec((BT, NUM_SHARED), lambda t, e: (t, 0)),
            pl.BlockSpec((BT, D_MODEL), lambda t, e: (t, 0)),
            pl.BlockSpec((1, D_MODEL, D_MODEL), lambda t, e: (e, 0, 0)),
        ],
        out_specs=pl.BlockSpec((BT, D_MODEL), lambda t, e: (t, 0)),
        out_shape=jax.ShapeDtypeStruct((N_TOK, D_MODEL), jnp.float32),
        scratch_shapes=[pltpu.VMEM((BT, 16), jnp.float32)],
        compiler_params=pltpu.CompilerParams(
            dimension_semantics=("parallel", "arbitrary"),
        ),
    )(spec_router_logits, shared_router_logits, x16, w_all)


# speedup vs baseline: 1.4441x; 1.0204x over previous
import jax
import jax.numpy as jnp
from jax.experimental import pallas as pl
from jax.experimental.pallas import tpu as pltpu

N_TOK = 4096
D_MODEL = 1024
NUM_SPEC = 8
NUM_SHARED = 2
NUM_TOTAL = NUM_SPEC + NUM_SHARED
BT = 2048


def _routing_body(slT_ref, shlT_ref, ct_ref):
    sl = slT_ref[...]
    m = jnp.max(sl, axis=0, keepdims=True)
    ex = jnp.exp(sl - m)
    z = jnp.sum(ex, axis=0, keepdims=True)
    sub8 = jax.lax.broadcasted_iota(jnp.int32, sl.shape, 0)
    g1 = jnp.max(ex, axis=0, keepdims=True)
    a1 = jnp.min(jnp.where(ex == g1, sub8, NUM_SPEC), axis=0, keepdims=True)
    ex2 = jnp.where(sub8 == a1, 0.0, ex)
    g2 = jnp.max(ex2, axis=0, keepdims=True)
    a2 = jnp.min(jnp.where(ex2 == g2, sub8, NUM_SPEC), axis=0, keepdims=True)
    denom = g1 + g2 + 1e-6 * z
    w1 = g1 / denom
    w2 = g2 / denom
    sub16 = jax.lax.broadcasted_iota(jnp.int32, (16, sl.shape[1]), 0)
    ct = w1 * (sub16 == a1).astype(jnp.float32) + w2 * (sub16 == a2).astype(jnp.float32)
    shl = shlT_ref[...]
    sm = jnp.max(shl, axis=0, keepdims=True)
    sex = jnp.exp(shl - sm)
    ssum = jnp.sum(sex, axis=0, keepdims=True)
    ct = ct + (sex[0:1, :] / ssum) * (sub16 == NUM_SPEC).astype(jnp.float32)
    ct = ct + (sex[1:2, :] / ssum) * (sub16 == NUM_SPEC + 1).astype(jnp.float32)
    ct_ref[...] = ct.astype(jnp.bfloat16)


def _moe_body(ct_ref, x_ref, w_ref, o_ref):
    e = pl.program_id(1)
    onehot_rows = (jax.lax.broadcasted_iota(jnp.int32, (16, D_MODEL), 0) == e).astype(jnp.bfloat16)
    scale_bc = jax.lax.dot_general(
        ct_ref[...], onehot_rows, (((0,), (0,)), ((), ())),
        preferred_element_type=jnp.float32)
    contrib = scale_bc * jnp.dot(x_ref[...], w_ref[0], preferred_element_type=jnp.float32)

    @pl.when(e == 0)
    def _init():
        o_ref[...] = contrib

    @pl.when(e > 0)
    def _acc():
        o_ref[...] += contrib


def kernel(x, spec_router_logits, shared_router_logits, spec_expert_weights, shared_expert_weights):
    w_all = jnp.concatenate([spec_expert_weights, shared_expert_weights], axis=0).astype(jnp.bfloat16)
    x16 = x.astype(jnp.bfloat16)
    ct = pl.pallas_call(
        _routing_body,
        out_shape=jax.ShapeDtypeStruct((16, N_TOK), jnp.bfloat16),
    )(spec_router_logits.T, shared_router_logits.T)
    grid = (N_TOK // BT, NUM_TOTAL)
    return pl.pallas_call(
        _moe_body,
        grid=grid,
        in_specs=[
            pl.BlockSpec((16, BT), lambda t, e: (0, t)),
            pl.BlockSpec((BT, D_MODEL), lambda t, e: (t, 0)),
            pl.BlockSpec((1, D_MODEL, D_MODEL), lambda t, e: (e, 0, 0)),
        ],
        out_specs=pl.BlockSpec((BT, D_MODEL), lambda t, e: (t, 0)),
        out_shape=jax.ShapeDtypeStruct((N_TOK, D_MODEL), jnp.float32),
        compiler_params=pltpu.CompilerParams(
            dimension_semantics=("parallel", "arbitrary"),
        ),
    )(ct, x16, w_all)


# speedup vs baseline: 1.4504x; 1.0044x over previous
import jax
import jax.numpy as jnp
from jax.experimental import pallas as pl
from jax.experimental.pallas import tpu as pltpu

N_TOK = 4096
D_MODEL = 1024
NUM_SPEC = 8
NUM_SHARED = 2
NUM_TOTAL = NUM_SPEC + NUM_SHARED
BT = 2048


def _routing_body(slT_ref, shlT_ref, ct_ref):
    sl = slT_ref[...]
    m = jnp.max(sl, axis=0, keepdims=True)
    ex = jnp.exp(sl - m)
    z = jnp.sum(ex, axis=0, keepdims=True)
    sub8 = jax.lax.broadcasted_iota(jnp.int32, sl.shape, 0)
    g1 = jnp.max(ex, axis=0, keepdims=True)
    a1 = jnp.min(jnp.where(ex == g1, sub8, NUM_SPEC), axis=0, keepdims=True)
    ex2 = jnp.where(sub8 == a1, 0.0, ex)
    g2 = jnp.max(ex2, axis=0, keepdims=True)
    a2 = jnp.min(jnp.where(ex2 == g2, sub8, NUM_SPEC), axis=0, keepdims=True)
    denom = g1 + g2 + 1e-6 * z
    w1 = g1 / denom
    w2 = g2 / denom
    sub16 = jax.lax.broadcasted_iota(jnp.int32, (16, sl.shape[1]), 0)
    ct = w1 * (sub16 == a1).astype(jnp.float32) + w2 * (sub16 == a2).astype(jnp.float32)
    shl = shlT_ref[...]
    sm = jnp.max(shl, axis=0, keepdims=True)
    sex = jnp.exp(shl - sm)
    ssum = jnp.sum(sex, axis=0, keepdims=True)
    ct = ct + (sex[0:1, :] / ssum) * (sub16 == NUM_SPEC).astype(jnp.float32)
    ct = ct + (sex[1:2, :] / ssum) * (sub16 == NUM_SPEC + 1).astype(jnp.float32)
    ct_ref[...] = ct.astype(jnp.bfloat16)


def _moe_body(ct_ref, x_ref, w_ref, o_ref):
    e = pl.program_id(1)
    onehot_rows = (jax.lax.broadcasted_iota(jnp.int32, (16, D_MODEL), 0) == e).astype(jnp.bfloat16)
    scale_bc = jax.lax.dot_general(
        ct_ref[...], onehot_rows, (((0,), (0,)), ((), ())),
        preferred_element_type=jnp.float32)
    xs = scale_bc.astype(jnp.bfloat16) * x_ref[...]
    contrib = jnp.dot(xs, w_ref[0], preferred_element_type=jnp.float32)

    @pl.when(e == 0)
    def _init():
        o_ref[...] = contrib

    @pl.when(e > 0)
    def _acc():
        o_ref[...] += contrib


def kernel(x, spec_router_logits, shared_router_logits, spec_expert_weights, shared_expert_weights):
    w_all = jnp.concatenate([spec_expert_weights, shared_expert_weights], axis=0).astype(jnp.bfloat16)
    x16 = x.astype(jnp.bfloat16)
    ct = pl.pallas_call(
        _routing_body,
        out_shape=jax.ShapeDtypeStruct((16, N_TOK), jnp.bfloat16),
    )(spec_router_logits.T, shared_router_logits.T)
    grid = (N_TOK // BT, NUM_TOTAL)
    return pl.pallas_call(
        _moe_body,
        grid=grid,
        in_specs=[
            pl.BlockSpec((16, BT), lambda t, e: (0, t)),
            pl.BlockSpec((BT, D_MODEL), lambda t, e: (t, 0)),
            pl.BlockSpec((1, D_MODEL, D_MODEL), lambda t, e: (e, 0, 0)),
        ],
        out_specs=pl.BlockSpec((BT, D_MODEL), lambda t, e: (t, 0)),
        out_shape=jax.ShapeDtypeStruct((N_TOK, D_MODEL), jnp.float32),
        compiler_params=pltpu.CompilerParams(
            dimension_semantics=("parallel", "arbitrary"),
        ),
    )(ct, x16, w_all)


# speedup vs baseline: 1.6532x; 1.1398x over previous
import jax
import jax.numpy as jnp
from jax.experimental import pallas as pl
from jax.experimental.pallas import tpu as pltpu

N_TOK = 4096
D_MODEL = 1024
NUM_SPEC = 8
NUM_SHARED = 2
NUM_TOTAL = NUM_SPEC + NUM_SHARED
BT = 2048


def _routing(slT, shlT):
    m = jnp.max(slT, axis=0, keepdims=True)
    ex = jnp.exp(slT - m)
    z = jnp.sum(ex, axis=0, keepdims=True)
    sub8 = jax.lax.broadcasted_iota(jnp.int32, slT.shape, 0)
    g1 = jnp.max(ex, axis=0, keepdims=True)
    a1 = jnp.min(jnp.where(ex == g1, sub8, NUM_SPEC), axis=0, keepdims=True)
    ex2 = jnp.where(sub8 == a1, 0.0, ex)
    g2 = jnp.max(ex2, axis=0, keepdims=True)
    a2 = jnp.min(jnp.where(ex2 == g2, sub8, NUM_SPEC), axis=0, keepdims=True)
    denom = g1 + g2 + 1e-6 * z
    w1 = g1 / denom
    w2 = g2 / denom
    sub16 = jax.lax.broadcasted_iota(jnp.int32, (16, slT.shape[1]), 0)
    ct = w1 * (sub16 == a1).astype(jnp.float32) + w2 * (sub16 == a2).astype(jnp.float32)
    sm = jnp.max(shlT, axis=0, keepdims=True)
    sex = jnp.exp(shlT - sm)
    ssum = jnp.sum(sex, axis=0, keepdims=True)
    ct = ct + (sex[0:1, :] / ssum) * (sub16 == NUM_SPEC).astype(jnp.float32)
    ct = ct + (sex[1:2, :] / ssum) * (sub16 == NUM_SPEC + 1).astype(jnp.float32)
    return ct


def _moe_body(slT_ref, shlT_ref, x_ref, wspec_ref, wshared_ref, o_ref, ct_ref):
    t = pl.program_id(0)
    e = pl.program_id(1)

    @pl.when((t == 0) & (e == 0))
    def _do_routing():
        ct_ref[...] = _routing(slT_ref[...], shlT_ref[...]).astype(jnp.bfloat16)

    onehot_rows = (jax.lax.broadcasted_iota(jnp.int32, (16, D_MODEL), 0) == e).astype(jnp.bfloat16)
    scale_bc = jax.lax.dot_general(
        ct_ref[:, pl.ds(t * BT, BT)], onehot_rows, (((0,), (0,)), ((), ())),
        preferred_element_type=jnp.float32)
    xs = scale_bc.astype(jnp.bfloat16) * x_ref[...]

    def _acc(w_ref):
        contrib = jnp.dot(xs, w_ref[0].astype(jnp.bfloat16), preferred_element_type=jnp.float32)

        @pl.when(e == 0)
        def _init():
            o_ref[...] = contrib

        @pl.when(e > 0)
        def _add():
            o_ref[...] += contrib

    @pl.when(e < NUM_SPEC)
    def _spec():
        _acc(wspec_ref)

    @pl.when(e >= NUM_SPEC)
    def _shared():
        _acc(wshared_ref)


def kernel(x, spec_router_logits, shared_router_logits, spec_expert_weights, shared_expert_weights):
    x16 = x.astype(jnp.bfloat16)
    return pl.pallas_call(
        _moe_body,
        grid=(N_TOK // BT, NUM_TOTAL),
        in_specs=[
            pl.BlockSpec((NUM_SPEC, N_TOK), lambda t, e: (0, 0)),
            pl.BlockSpec((NUM_SHARED, N_TOK), lambda t, e: (0, 0)),
            pl.BlockSpec((BT, D_MODEL), lambda t, e: (t, 0)),
            pl.BlockSpec((1, D_MODEL, D_MODEL), lambda t, e: (jnp.minimum(e, NUM_SPEC - 1), 0, 0)),
            pl.BlockSpec((1, D_MODEL, D_MODEL), lambda t, e: (jnp.maximum(e - NUM_SPEC, 0), 0, 0)),
        ],
        out_specs=pl.BlockSpec((BT, D_MODEL), lambda t, e: (t, 0)),
        out_shape=jax.ShapeDtypeStruct((N_TOK, D_MODEL), jnp.float32),
        scratch_shapes=[pltpu.VMEM((16, N_TOK), jnp.bfloat16)],
        compiler_params=pltpu.CompilerParams(
            dimension_semantics=("arbitrary", "arbitrary"),
        ),
    )(spec_router_logits.T, shared_router_logits.T, x16, spec_expert_weights, shared_expert_weights)
